# baseline (speedup 1.0000x reference)
import jax
import jax.numpy as jnp
from jax import lax
from jax.experimental import pallas as pl
from jax.experimental.pallas import tpu as pltpu
from jax.experimental.pallas import tpu_sc as plsc

D = 128
NC = 2
NS = 16
NW = NC * NS
K = 88
SLOTS = 4
CH = 4
RB = 1000


def _mm_relu1_body(x_ref, w0_ref, b0_ref, w1_ref, b1_ref, o0_ref, o1_ref):
    y = jnp.maximum(x_ref[...], 0.0)
    o0_ref[...] = jnp.dot(y, w0_ref[...], preferred_element_type=jnp.float32) + b0_ref[...]
    o1_ref[...] = jnp.dot(y, w1_ref[...], preferred_element_type=jnp.float32) + b1_ref[...]


def _mm_relu2_body(p0_ref, p1_ref, w0_ref, b0_ref, w1_ref, b1_ref, o0_ref, o1_ref):
    y = jnp.maximum(p0_ref[...] + p1_ref[...], 0.0)
    o0_ref[...] = jnp.dot(y, w0_ref[...], preferred_element_type=jnp.float32) + b0_ref[...]
    o1_ref[...] = jnp.dot(y, w1_ref[...], preferred_element_type=jnp.float32) + b1_ref[...]


def _add_body(p0_ref, p1_ref, o_ref):
    o_ref[...] = p0_ref[...] + p1_ref[...]


def _make_mm(body, n_data_in, n):
    grid = (n // RB,)
    data_spec = pl.BlockSpec((RB, D), lambda i: (i, 0))
    w_spec = pl.BlockSpec((D, D), lambda i: (0, 0))
    b_spec = pl.BlockSpec((1, D), lambda i: (0, 0))
    in_specs = [data_spec] * n_data_in + [w_spec, b_spec, w_spec, b_spec]
    return pl.pallas_call(
        body,
        grid=grid,
        in_specs=in_specs,
        out_specs=[data_spec, data_spec],
        out_shape=[jax.ShapeDtypeStruct((n, D), jnp.float32)] * 2,
    )


def _make_add(n):
    spec = pl.BlockSpec((RB, D), lambda i: (i, 0))
    return pl.pallas_call(
        _add_body,
        grid=(n // RB,),
        in_specs=[spec, spec],
        out_specs=spec,
        out_shape=jax.ShapeDtypeStruct((n, D), jnp.float32),
    )


def _make_scatter(n, ngroups, npad):
    base = (n // NS) // 8 * 8
    tail = n - base * NS
    mesh = plsc.VectorSubcoreMesh(
        core_axis_name="c", subcore_axis_name="s",
        num_cores=NC, num_subcores=NS)

    def body(idx_hbm, table_hbm, init0_hbm, init1_hbm, out_hbm,
             acc, idx_v, rows0, rows1, rows2, rows3,
             sem0, sem1, sem2, sem3, semi0, semi1):
        c = lax.axis_index("c")
        s = lax.axis_index("s")
        wid = c * NS + s
        off = pl.multiple_of(s * base, 8)
        stripe = pl.ds(off, base)
        tail_stripe = pl.ds(n - tail, tail)
        rows = (rows0, rows1, rows2, rows3)
        sems = (sem0, sem1, sem2, sem3)
        semis = (semi0, semi1)

        def stripe_copy(src, dst):
            pltpu.sync_copy(src.at[stripe], dst.at[stripe])
            if tail:
                @pl.when(s == NS - 1)
                def _():
                    pltpu.sync_copy(src.at[tail_stripe], dst.at[tail_stripe])

        @pl.when(c == 0)
        def _():
            stripe_copy(init0_hbm, acc)

        @pl.when(c != 0)
        def _():
            stripe_copy(init1_hbm, acc)

        def gather(slot, cc, b):
            return pltpu.async_copy(
                table_hbm.at[idx_v.at[slot, cc, 0]], rows[b], sems[b])

        pltpu.sync_copy(idx_hbm.at[wid, 0], idx_v.at[0])
        for b in range(SLOTS):
            gather(0, b, b)
        plsc.subcore_barrier()

        def pair(p, carry):
            for a in (0, 1):
                h = 2 * p + a

                @pl.when(h + 1 < ngroups)
                def _():
                    pltpu.async_copy(idx_hbm.at[wid, h + 1], idx_v.at[1 - a],
                                     semis[1 - a])

                for cc in range(CH):
                    b = cc % SLOTS
                    pltpu.make_async_copy(
                        table_hbm.at[idx_v.at[a, cc, 0]], rows[b], sems[b]
                    ).wait()
                    pltpu.sync_copy(rows[b], acc.at[idx_v.at[a, cc, 1]],
                                    add=True)
                    nxt = cc + SLOTS
                    if nxt < CH:
                        gather(a, nxt, b)
                    else:
                        @pl.when(h + 1 < ngroups)
                        def _():
                            if cc == CH - SLOTS:
                                pltpu.make_async_copy(
                                    idx_hbm.at[wid, h + 1], idx_v.at[1 - a],
                                    semis[1 - a]).wait()
                            gather(1 - a, nxt - CH, b)
            return carry

        lax.fori_loop(0, ngroups // 2, pair, 0)
        plsc.subcore_barrier()
        stripe_copy(acc, out_hbm.at[c])

    return pl.kernel(
        body,
        out_type=jax.ShapeDtypeStruct((NC, n, D), jnp.float32),
        mesh=mesh,
        scratch_types=(
            [pltpu.VMEM_SHARED((npad, D), jnp.float32),
             pltpu.VMEM((2, CH, 2, K), jnp.int32)]
            + [pltpu.VMEM((K, D), jnp.float32)] * SLOTS
            + [pltpu.SemaphoreType.DMA] * (SLOTS + 2)
        ),
    )


def kernel(x, edge, w0_1, b0_1, w1_1, b1_1, w0_2, b0_2, w1_2, b1_2):
    n = x.shape[0]
    e = edge.shape[0]

    src = edge[:, 0]
    dst = edge[:, 1]
    gidx = jnp.concatenate([dst, src])
    sidx = jnp.concatenate([src, dst])
    per_w = -(-(2 * e) // NW)
    nchunk = -(-per_w // K)
    nchunk = -(-nchunk // (2 * CH)) * (2 * CH)
    ngroups = nchunk // CH
    pad = NW * nchunk * K - 2 * e
    gidx = jnp.concatenate([gidx, jnp.zeros((pad,), jnp.int32)])
    sidx = jnp.concatenate(
        [sidx, n + (jnp.arange(pad, dtype=jnp.int32) % 8)])
    idx = jnp.stack([gidx.reshape(NW, ngroups, CH, K),
                     sidx.reshape(NW, ngroups, CH, K)], axis=3)

    mm1 = _make_mm(_mm_relu1_body, 1, n)
    mm2 = _make_mm(_mm_relu2_body, 2, n)
    scat = _make_scatter(n, ngroups, n + 8)
    addk = _make_add(n)

    zeros = jnp.zeros((n, D), jnp.float32)
    b0_1r = b0_1.reshape(1, D)
    b1_1r = b1_1.reshape(1, D)
    b0_2r = b0_2.reshape(1, D)
    b1_2r = b1_2.reshape(1, D)

    a1, t1 = mm1(x, w0_1.T, b0_1r, w1_1.T, b1_1r)
    p1 = scat(idx, t1, a1, zeros)
    a2, t2 = mm2(p1[0], p1[1], w0_2.T, b0_2r, w1_2.T, b1_2r)
    p2 = scat(idx, t2, a2, x)
    return addk(p2[0], p2[1])


# speedup vs baseline: 1.6423x; 1.6423x over previous
import jax
import jax.numpy as jnp
from jax import lax
from jax.experimental import pallas as pl
from jax.experimental.pallas import tpu as pltpu
from jax.experimental.pallas import tpu_sc as plsc

D = 128
NC = 2
NS = 16
NW = NC * NS
K = 120
SLOTS = 3
CH = 6
RB = 1000


def _mm_relu1_body(x_ref, w0_ref, b0_ref, w1_ref, b1_ref, o0_ref, o1_ref):
    y = jnp.maximum(x_ref[...], 0.0)
    o0_ref[...] = jnp.dot(y, w0_ref[...], preferred_element_type=jnp.float32) + b0_ref[...]
    o1_ref[...] = jnp.dot(y, w1_ref[...], preferred_element_type=jnp.float32) + b1_ref[...]


def _mm_relu2_body(p0_ref, p1_ref, w0_ref, b0_ref, w1_ref, b1_ref, o0_ref, o1_ref):
    y = jnp.maximum(p0_ref[...] + p1_ref[...], 0.0)
    o0_ref[...] = jnp.dot(y, w0_ref[...], preferred_element_type=jnp.float32) + b0_ref[...]
    o1_ref[...] = jnp.dot(y, w1_ref[...], preferred_element_type=jnp.float32) + b1_ref[...]


def _add_body(p0_ref, p1_ref, o_ref):
    o_ref[...] = p0_ref[...] + p1_ref[...]


def _make_mm(body, n_data_in, n):
    grid = (n // RB,)
    data_spec = pl.BlockSpec((RB, D), lambda i: (i, 0))
    w_spec = pl.BlockSpec((D, D), lambda i: (0, 0))
    b_spec = pl.BlockSpec((1, D), lambda i: (0, 0))
    in_specs = [data_spec] * n_data_in + [w_spec, b_spec, w_spec, b_spec]
    return pl.pallas_call(
        body,
        grid=grid,
        in_specs=in_specs,
        out_specs=[data_spec, data_spec],
        out_shape=[jax.ShapeDtypeStruct((n, D), jnp.float32)] * 2,
    )


def _make_add(n):
    spec = pl.BlockSpec((RB, D), lambda i: (i, 0))
    return pl.pallas_call(
        _add_body,
        grid=(n // RB,),
        in_specs=[spec, spec],
        out_specs=spec,
        out_shape=jax.ShapeDtypeStruct((n, D), jnp.float32),
    )


def _make_scatter(n, ngroups, npad):
    base = (n // NS) // 8 * 8
    tail = n - base * NS
    mesh = plsc.VectorSubcoreMesh(
        core_axis_name="c", subcore_axis_name="s",
        num_cores=NC, num_subcores=NS)

    def body(idx_hbm, table_hbm, init0_hbm, init1_hbm, out_hbm,
             acc, idx_v, rows0, rows1, rows2,
             sem0, sem1, sem2, semi0, semi1):
        c = lax.axis_index("c")
        s = lax.axis_index("s")
        wid = c * NS + s
        off = pl.multiple_of(s * base, 8)
        stripe = pl.ds(off, base)
        tail_stripe = pl.ds(n - tail, tail)
        rows = (rows0, rows1, rows2)
        sems = (sem0, sem1, sem2)
        semis = (semi0, semi1)

        def stripe_copy(src, dst):
            pltpu.sync_copy(src.at[stripe], dst.at[stripe])
            if tail:
                @pl.when(s == NS - 1)
                def _():
                    pltpu.sync_copy(src.at[tail_stripe], dst.at[tail_stripe])

        @pl.when(c == 0)
        def _():
            stripe_copy(init0_hbm, acc)

        @pl.when(c != 0)
        def _():
            stripe_copy(init1_hbm, acc)

        def gather(slot, cc, b):
            return pltpu.async_copy(
                table_hbm.at[idx_v.at[slot, cc, 0]], rows[b], sems[b])

        pltpu.sync_copy(idx_hbm.at[wid, 0], idx_v.at[0])
        for b in range(SLOTS):
            gather(0, b, b)
        plsc.subcore_barrier()

        def pair(p, carry):
            for a in (0, 1):
                h = 2 * p + a

                @pl.when(h + 1 < ngroups)
                def _():
                    pltpu.async_copy(idx_hbm.at[wid, h + 1], idx_v.at[1 - a],
                                     semis[1 - a])

                for cc in range(CH):
                    b = cc % SLOTS
                    pltpu.make_async_copy(
                        table_hbm.at[idx_v.at[a, cc, 0]], rows[b], sems[b]
                    ).wait()
                    pltpu.sync_copy(rows[b], acc.at[idx_v.at[a, cc, 1]],
                                    add=True)
                    nxt = cc + SLOTS
                    if nxt < CH:
                        gather(a, nxt, b)
                    else:
                        @pl.when(h + 1 < ngroups)
                        def _():
                            if cc == CH - SLOTS:
                                pltpu.make_async_copy(
                                    idx_hbm.at[wid, h + 1], idx_v.at[1 - a],
                                    semis[1 - a]).wait()
                            gather(1 - a, nxt - CH, b)
            return carry

        lax.fori_loop(0, ngroups // 2, pair, 0)
        plsc.subcore_barrier()
        stripe_copy(acc, out_hbm.at[c])

    return pl.kernel(
        body,
        out_type=jax.ShapeDtypeStruct((NC, n, D), jnp.float32),
        mesh=mesh,
        scratch_types=(
            [pltpu.VMEM_SHARED((npad, D), jnp.float32),
             pltpu.VMEM((2, CH, 2, K), jnp.int32)]
            + [pltpu.VMEM((K, D), jnp.float32)] * SLOTS
            + [pltpu.SemaphoreType.DMA] * (SLOTS + 2)
        ),
    )


def kernel(x, edge, w0_1, b0_1, w1_1, b1_1, w0_2, b0_2, w1_2, b1_2):
    n = x.shape[0]
    e = edge.shape[0]

    src = edge[:, 0]
    dst = edge[:, 1]
    gidx = jnp.concatenate([dst, src])
    sidx = jnp.concatenate([src, dst])
    per_w = -(-(2 * e) // NW)
    nchunk = -(-per_w // K)
    nchunk = -(-nchunk // (2 * CH)) * (2 * CH)
    ngroups = nchunk // CH
    pad = NW * nchunk * K - 2 * e
    gidx = jnp.concatenate([gidx, jnp.zeros((pad,), jnp.int32)])
    sidx = jnp.concatenate(
        [sidx, n + (jnp.arange(pad, dtype=jnp.int32) % 8)])
    idx = jnp.stack([gidx.reshape(NW, ngroups, CH, K),
                     sidx.reshape(NW, ngroups, CH, K)], axis=3)

    mm1 = _make_mm(_mm_relu1_body, 1, n)
    mm2 = _make_mm(_mm_relu2_body, 2, n)
    scat = _make_scatter(n, ngroups, n + 8)
    addk = _make_add(n)

    zeros = jnp.zeros((n, D), jnp.float32)
    b0_1r = b0_1.reshape(1, D)
    b1_1r = b1_1.reshape(1, D)
    b0_2r = b0_2.reshape(1, D)
    b1_2r = b1_2.reshape(1, D)

    a1, t1 = mm1(x, w0_1.T, b0_1r, w1_1.T, b1_1r)
    p1 = scat(idx, t1, a1, zeros)
    a2, t2 = mm2(p1[0], p1[1], w0_2.T, b0_2r, w1_2.T, b1_2r)
    p2 = scat(idx, t2, a2, x)
    return addk(p2[0], p2[1])
